# initial kernel scaffold (unmeasured)
import jax
import jax.numpy as jnp
from jax import lax
from jax.experimental import pallas as pl
from jax.experimental.pallas import tpu as pltpu

N_Y = 4
B, S, D = 2, 512, 2048
H, DH, DR = 16, 128, 32
DC_SH = 128
BS = B * S
SCALE = (DH + DR) ** -0.5


def _mm(a, b, bn=1024):
    m, k = a.shape
    _, n = b.shape
    bn = min(bn, n)

    def body(a_ref, b_ref, o_ref):
        o_ref[:, :] = jnp.dot(a_ref[:, :], b_ref[:, :],
                              preferred_element_type=jnp.float32)

    return pl.pallas_call(
        body,
        grid=(n // bn,),
        in_specs=[
            pl.BlockSpec((m, k), lambda j: (0, 0)),
            pl.BlockSpec((k, bn), lambda j: (0, j)),
        ],
        out_specs=pl.BlockSpec((m, bn), lambda j: (0, j)),
        out_shape=jax.ShapeDtypeStruct((m, n), jnp.float32),
    )(a, b)


def _mm_acc4(c4, w4, bn=1024):
    _, m, k = c4.shape
    _, _, n = w4.shape
    bn = min(bn, n)

    def body(c_ref, w_ref, o_ref):
        y = pl.program_id(1)

        @pl.when(y == 0)
        def _():
            o_ref[:, :] = jnp.zeros_like(o_ref)

        o_ref[:, :] += jnp.dot(c_ref[0], w_ref[0],
                               preferred_element_type=jnp.float32)

    return pl.pallas_call(
        body,
        grid=(n // bn, N_Y),
        in_specs=[
            pl.BlockSpec((1, m, k), lambda j, y: (y, 0, 0)),
            pl.BlockSpec((1, k, bn), lambda j, y: (y, 0, j)),
        ],
        out_specs=pl.BlockSpec((m, bn), lambda j, y: (0, j)),
        out_shape=jax.ShapeDtypeStruct((m, n), jnp.float32),
    )(c4, w4)


def _ring_allgather(c_sh, wuk_sh, wuv_sh):

    def body(cs_ref, wk_ref, wv_ref, cf_ref, wkf_ref, wvf_ref,
             send_sems, recv_sems):
        xi = lax.axis_index("x")
        my = lax.axis_index("y")
        zi = lax.axis_index("z")
        left = lax.rem(my + N_Y - 1, N_Y)
        right = lax.rem(my + 1, N_Y)

        barrier = pltpu.get_barrier_semaphore()
        pl.semaphore_signal(barrier, inc=1, device_id=(xi, left, zi),
                            device_id_type=pl.DeviceIdType.MESH)
        pl.semaphore_signal(barrier, inc=1, device_id=(xi, right, zi),
                            device_id_type=pl.DeviceIdType.MESH)
        pl.semaphore_wait(barrier, 2)

        cf_ref[my] = cs_ref[:, :]
        wkf_ref[my] = wk_ref[:, :]
        wvf_ref[my] = wv_ref[:, :]

        for h in range(N_Y - 1):
            origin = lax.rem(my + N_Y - h, N_Y)
            rdmas = []
            for t, ref in enumerate((cf_ref, wkf_ref, wvf_ref)):
                src = ref.at[origin]
                rdma = pltpu.make_async_remote_copy(
                    src_ref=src,
                    dst_ref=src,
                    send_sem=send_sems.at[t, h],
                    recv_sem=recv_sems.at[t, h],
                    device_id=(xi, right, zi),
                    device_id_type=pl.DeviceIdType.MESH,
                )
                rdma.start()
                rdmas.append(rdma)
            for rdma in rdmas:
                rdma.wait()

    return pl.pallas_call(
        body,
        in_specs=[pl.BlockSpec(memory_space=pltpu.VMEM)] * 3,
        out_specs=[pl.BlockSpec(memory_space=pltpu.VMEM)] * 3,
        out_shape=[
            jax.ShapeDtypeStruct((N_Y, BS, DC_SH), jnp.float32),
            jax.ShapeDtypeStruct((N_Y, DC_SH, D), jnp.float32),
            jax.ShapeDtypeStruct((N_Y, DC_SH, D), jnp.float32),
        ],
        scratch_shapes=[
            pltpu.SemaphoreType.DMA((3, N_Y - 1)),
            pltpu.SemaphoreType.DMA((3, N_Y - 1)),
        ],
        compiler_params=pltpu.CompilerParams(collective_id=0),
    )(c_sh, wuk_sh, wuv_sh)


def _attention(Q, K, V, Qr, Kr):

    def body(q_ref, k_ref, v_ref, qr_ref, kr_ref, o_ref):
        q = q_ref[:, :]
        k = k_ref[:, :]
        v = v_ref[:, :]
        qr = qr_ref[:, :]
        kr = kr_ref[:, :]
        s = lax.dot_general(q, k, (((1,), (1,)), ((), ())),
                            preferred_element_type=jnp.float32)
        s = s + lax.dot_general(qr, kr, (((1,), (1,)), ((), ())),
                                preferred_element_type=jnp.float32)
        s = s * SCALE
        m = jnp.max(s, axis=-1, keepdims=True)
        p = jnp.exp(s - m)
        p = p / jnp.sum(p, axis=-1, keepdims=True)
        o_ref[:, :] = jnp.dot(p, v, preferred_element_type=jnp.float32)

    return pl.pallas_call(
        body,
        grid=(B, H),
        in_specs=[
            pl.BlockSpec((S, DH), lambda b, h: (b, h)),
            pl.BlockSpec((S, DH), lambda b, h: (b, h)),
            pl.BlockSpec((S, DH), lambda b, h: (b, h)),
            pl.BlockSpec((S, DR), lambda b, h: (b, h)),
            pl.BlockSpec((S, DR), lambda b, h: (b, 0)),
        ],
        out_specs=pl.BlockSpec((S, DH), lambda b, h: (b, h)),
        out_shape=jax.ShapeDtypeStruct((BS, D), jnp.float32),
    )(Q, K, V, Qr, Kr)


def kernel(x, Wdkv, Wuk, Wuv, Wq, Wqr, Wkr, Wo):
    x2 = x.reshape(BS, D)
    c_sh = _mm(x2, Wdkv, bn=DC_SH)
    c4, wuk4, wuv4 = _ring_allgather(c_sh, Wuk, Wuv)
    K = _mm_acc4(c4, wuk4)
    V = _mm_acc4(c4, wuv4)
    Q = _mm(x2, Wq)
    Qr = _mm(x2, Wqr, bn=512)
    Kr = _mm(x2, Wkr, bn=DR)
    O = _attention(Q, K, V, Qr, Kr)
    out = _mm(O, Wo)
    return out.reshape(B, S, D)


# baseline (device time: 198028 ns/iter reference)
import jax
import jax.numpy as jnp
from jax import lax
from jax.experimental import pallas as pl
from jax.experimental.pallas import tpu as pltpu

N_Y = 4
B, S, D = 2, 512, 2048
H, DH, DR = 16, 128, 32
DC_SH = 128
BS = B * S
SCALE = (DH + DR) ** -0.5


def _mm(a, b, bn=1024):
    m, k = a.shape
    _, n = b.shape
    bn = min(bn, n)

    def body(a_ref, b_ref, o_ref):
        o_ref[:, :] = jnp.dot(a_ref[:, :], b_ref[:, :],
                              preferred_element_type=jnp.float32)

    return pl.pallas_call(
        body,
        grid=(n // bn,),
        in_specs=[
            pl.BlockSpec((m, k), lambda j: (0, 0)),
            pl.BlockSpec((k, bn), lambda j: (0, j)),
        ],
        out_specs=pl.BlockSpec((m, bn), lambda j: (0, j)),
        out_shape=jax.ShapeDtypeStruct((m, n), jnp.float32),
    )(a, b)


def _mm_acc4(c4, w4, bn=1024):
    _, m, k = c4.shape
    _, _, n = w4.shape
    bn = min(bn, n)

    def body(c_ref, w_ref, o_ref):
        y = pl.program_id(1)

        @pl.when(y == 0)
        def _():
            o_ref[:, :] = jnp.zeros_like(o_ref)

        o_ref[:, :] += jnp.dot(c_ref[0], w_ref[0],
                               preferred_element_type=jnp.float32)

    return pl.pallas_call(
        body,
        grid=(n // bn, N_Y),
        in_specs=[
            pl.BlockSpec((1, m, k), lambda j, y: (y, 0, 0)),
            pl.BlockSpec((1, k, bn), lambda j, y: (y, 0, j)),
        ],
        out_specs=pl.BlockSpec((m, bn), lambda j, y: (0, j)),
        out_shape=jax.ShapeDtypeStruct((m, n), jnp.float32),
    )(c4, w4)


def _mm_tt(w, a):
    k, n = w.shape
    m, _ = a.shape

    def body(w_ref, a_ref, o_ref):
        o_ref[:, :] = lax.dot_general(
            w_ref[:, :], a_ref[:, :], (((0,), (1,)), ((), ())),
            preferred_element_type=jnp.float32)

    return pl.pallas_call(
        body,
        in_specs=[pl.BlockSpec(memory_space=pltpu.VMEM)] * 2,
        out_specs=pl.BlockSpec(memory_space=pltpu.VMEM),
        out_shape=jax.ShapeDtypeStruct((n, m), jnp.float32),
    )(w, a)


def _ring_allgather(c_sh, wuk_sh, wuv_sh):

    def body(cs_ref, wk_ref, wv_ref, cf_ref, wkf_ref, wvf_ref,
             send_sems, recv_sems):
        xi = lax.axis_index("x")
        my = lax.axis_index("y")
        zi = lax.axis_index("z")
        left = lax.rem(my + N_Y - 1, N_Y)
        right = lax.rem(my + 1, N_Y)

        barrier = pltpu.get_barrier_semaphore()
        pl.semaphore_signal(barrier, inc=1, device_id=(xi, left, zi),
                            device_id_type=pl.DeviceIdType.MESH)
        pl.semaphore_signal(barrier, inc=1, device_id=(xi, right, zi),
                            device_id_type=pl.DeviceIdType.MESH)
        pl.semaphore_wait(barrier, 2)

        cf_ref[my] = cs_ref[:, :]
        wkf_ref[my] = wk_ref[:, :]
        wvf_ref[my] = wv_ref[:, :]

        for h in range(N_Y - 1):
            origin = lax.rem(my + N_Y - h, N_Y)
            rdmas = []
            for t, ref in enumerate((cf_ref, wkf_ref, wvf_ref)):
                src = ref.at[origin]
                rdma = pltpu.make_async_remote_copy(
                    src_ref=src,
                    dst_ref=src,
                    send_sem=send_sems.at[t, h],
                    recv_sem=recv_sems.at[t, h],
                    device_id=(xi, right, zi),
                    device_id_type=pl.DeviceIdType.MESH,
                )
                rdma.start()
                rdmas.append(rdma)
            for rdma in rdmas:
                rdma.wait()

    return pl.pallas_call(
        body,
        in_specs=[pl.BlockSpec(memory_space=pltpu.VMEM)] * 3,
        out_specs=[pl.BlockSpec(memory_space=pltpu.VMEM)] * 3,
        out_shape=[
            jax.ShapeDtypeStruct((N_Y, BS, DC_SH), jnp.float32),
            jax.ShapeDtypeStruct((N_Y, DC_SH, D), jnp.float32),
            jax.ShapeDtypeStruct((N_Y, DC_SH, D), jnp.float32),
        ],
        scratch_shapes=[
            pltpu.SemaphoreType.DMA((3, N_Y - 1)),
            pltpu.SemaphoreType.DMA((3, N_Y - 1)),
        ],
        compiler_params=pltpu.CompilerParams(collective_id=0),
    )(c_sh, wuk_sh, wuv_sh)


def _attention(Q, K, V, QrT, Kr):

    def body(q_ref, k_ref, v_ref, qr_ref, kr_ref, o_ref):
        q = q_ref[:, :]
        k = k_ref[:, :]
        v = v_ref[:, :]
        qr_t = qr_ref[:, :]
        kr = kr_ref[:, :]
        s = lax.dot_general(q, k, (((1,), (1,)), ((), ())),
                            preferred_element_type=jnp.float32)
        s = s + lax.dot_general(qr_t, kr, (((0,), (1,)), ((), ())),
                                preferred_element_type=jnp.float32)
        s = s * SCALE
        m = jnp.max(s, axis=-1, keepdims=True)
        p = jnp.exp(s - m)
        p = p / jnp.sum(p, axis=-1, keepdims=True)
        o_ref[:, :] = jnp.dot(p, v, preferred_element_type=jnp.float32)

    return pl.pallas_call(
        body,
        grid=(B, H),
        in_specs=[
            pl.BlockSpec((S, DH), lambda b, h: (b, h)),
            pl.BlockSpec((S, DH), lambda b, h: (b, h)),
            pl.BlockSpec((S, DH), lambda b, h: (b, h)),
            pl.BlockSpec((DR, S), lambda b, h: (h, b)),
            pl.BlockSpec((S, DR), lambda b, h: (b, 0)),
        ],
        out_specs=pl.BlockSpec((S, DH), lambda b, h: (b, h)),
        out_shape=jax.ShapeDtypeStruct((BS, D), jnp.float32),
    )(Q, K, V, QrT, Kr)


def kernel(x, Wdkv, Wuk, Wuv, Wq, Wqr, Wkr, Wo):
    x2 = x.reshape(BS, D)
    c_sh = _mm(x2, Wdkv, bn=DC_SH)
    c4, wuk4, wuv4 = _ring_allgather(c_sh, Wuk, Wuv)
    K = _mm_acc4(c4, wuk4)
    V = _mm_acc4(c4, wuv4)
    Q = _mm(x2, Wq)
    QrT = _mm_tt(Wqr, x2)
    Kr = _mm(x2, Wkr, bn=DR)
    O = _attention(Q, K, V, QrT, Kr)
    out = _mm(O, Wo)
    return out.reshape(B, S, D)
